# baseline (speedup 1.0000x reference)
import jax
import jax.numpy as jnp
from jax.experimental import pallas as pl
from jax.experimental.pallas import tpu as pltpu

_R, _C = 160, 128
_P = _R * _C
_G = _R // 8
_MOUT = 200
_BIG = 2**30
_B = 16


def _tile_iota():
    return (jax.lax.broadcasted_iota(jnp.int32, (8, _C), 0) * _C
            + jax.lax.broadcasted_iota(jnp.int32, (8, _C), 1))


def _combine(a, b):
    take_b = (b[0] > a[0]) | ((b[0] == a[0]) & (b[1] < a[1]))
    return (jnp.where(take_b, b[0], a[0]), jnp.where(take_b, b[1], a[1]))


def _nms_kernel(thr_ref, x1s, y1s, x2s, y2s, s, sel_ref, num_ref, ws):
    iou_thr = thr_ref[0, 0]
    score_thr = thr_ref[1, 0]
    ws[...] = jnp.where(s[...] > score_thr, s[...], -jnp.inf)

    ti = _tile_iota()

    def tree_sweep(idx, valid):
        accs = [None] * 4
        for g in range(_G):
            gs = pl.ds(g * 8, 8)
            w_g = ws[gs, :]
            if idx is not None:
                pick_g = ((ti + g * 1024) == idx) & valid
                w_g = jnp.where(pick_g, -jnp.inf, w_g)
                ws[gs, :] = w_g
            node = (w_g, ti + g * 1024)
            k = g % 4
            accs[k] = node if accs[k] is None else _combine(accs[k], node)
        return _combine(_combine(accs[0], accs[1]),
                        _combine(accs[2], accs[3]))

    m_v0, i_v0 = tree_sweep(None, None)

    def round_fn(state):
        num, stop, sel, sx1, sy1, sx2, sy2, sa, m_v, i_v = state
        m = jnp.max(m_v)
        valid = m > -jnp.inf
        idx = jnp.min(jnp.where(m_v == m, i_v, _BIG))
        r = idx // _C
        c = idx - r * _C
        zero = jnp.float32(0.0)
        b0 = x1s[r, c]
        b1 = y1s[r, c]
        b2 = x2s[r, c]
        b3 = y2s[r, c]
        a = (b2 - b0) * (b3 - b1)
        nm_v, ni_v = tree_sweep(idx, valid)
        xx1 = jnp.maximum(b0, sx1)
        yy1 = jnp.maximum(b1, sy1)
        xx2 = jnp.minimum(b2, sx2)
        yy2 = jnp.minimum(b3, sy2)
        inter = (jnp.clip(xx2 - xx1, 0.0, None)
                 * jnp.clip(yy2 - yy1, 0.0, None))
        union = jnp.maximum(a + sa - inter, 1e-6)
        iou = inter / union
        hit = (iou >= iou_thr) & (ti < num)
        hitcnt = jnp.sum(jnp.where(hit, 1.0, zero))
        accepted = valid & (hitcnt == zero) & (num < _MOUT)
        slot = accepted & (ti == num)
        sel = jnp.where(slot, idx, sel)
        sx1 = jnp.where(slot, b0, sx1)
        sy1 = jnp.where(slot, b1, sy1)
        sx2 = jnp.where(slot, b2, sx2)
        sy2 = jnp.where(slot, b3, sy2)
        sa = jnp.where(slot, a, sa)
        num = num + accepted.astype(jnp.int32)
        stop = jnp.logical_not(valid)
        return (num, stop, sel, sx1, sy1, sx2, sy2, sa, nm_v, ni_v)

    def cond(carry):
        return jnp.logical_and(carry[0] < _MOUT, jnp.logical_not(carry[1]))

    def body(carry):
        state = carry
        for _ in range(_B):
            state = round_fn(state)
        return state

    zf = jnp.zeros((8, _C), jnp.float32)
    carry = (jnp.int32(0), jnp.bool_(False),
             jnp.zeros((8, _C), jnp.int32), zf, zf, zf, zf, zf, m_v0, i_v0)
    carry = jax.lax.while_loop(cond, body, carry)
    sel_ref[...] = carry[2]
    num_ref[0, 0] = carry[0]


def kernel(boxes, scores, max_output_size, iou_threshold, scores_threshold):
    boxes = boxes.astype(jnp.float32)
    scores = scores.astype(jnp.float32)
    n = boxes.shape[0]
    pad = _P - n
    bx = jnp.pad(boxes, ((0, pad), (0, 0)))
    planes = bx.T.reshape(4, _R, _C)
    s = jnp.pad(scores, (0, pad), constant_values=-jnp.inf).reshape(_R, _C)
    thr = jnp.stack([jnp.asarray(iou_threshold, jnp.float32),
                     jnp.asarray(scores_threshold, jnp.float32)]).reshape(2, 1)

    sel_m, num_m = pl.pallas_call(
        _nms_kernel,
        in_specs=[
            pl.BlockSpec(memory_space=pltpu.SMEM),
            pl.BlockSpec(memory_space=pltpu.SMEM),
            pl.BlockSpec(memory_space=pltpu.SMEM),
            pl.BlockSpec(memory_space=pltpu.SMEM),
            pl.BlockSpec(memory_space=pltpu.SMEM),
            pl.BlockSpec(memory_space=pltpu.VMEM),
        ],
        out_specs=[
            pl.BlockSpec(memory_space=pltpu.VMEM),
            pl.BlockSpec(memory_space=pltpu.SMEM),
        ],
        out_shape=[
            jax.ShapeDtypeStruct((8, _C), jnp.int32),
            jax.ShapeDtypeStruct((1, 1), jnp.int32),
        ],
        scratch_shapes=[
            pltpu.VMEM((_R, _C), jnp.float32),
        ],
    )(thr, planes[0], planes[1], planes[2], planes[3], s)

    sel = sel_m.reshape(-1)[:_MOUT]
    num = jnp.minimum(num_m[0, 0], jnp.asarray(max_output_size, jnp.int32))
    return (sel, num)


# speedup vs baseline: 1.0029x; 1.0029x over previous
import jax
import jax.numpy as jnp
from jax.experimental import pallas as pl
from jax.experimental.pallas import tpu as pltpu

_R, _C = 160, 128
_P = _R * _C
_G = _R // 8
_MOUT = 200
_BIG = 2**30
_B = 16


def _tile_iota():
    return (jax.lax.broadcasted_iota(jnp.int32, (8, _C), 0) * _C
            + jax.lax.broadcasted_iota(jnp.int32, (8, _C), 1))


def _combine(a, b):
    take_b = (b[0] > a[0]) | ((b[0] == a[0]) & (b[1] < a[1]))
    return (jnp.where(take_b, b[0], a[0]), jnp.where(take_b, b[1], a[1]))


def _nms_kernel(thr_ref, x1s, y1s, x2s, y2s, s, sel_ref, num_ref, ws):
    iou_thr = thr_ref[0, 0]
    score_thr = thr_ref[1, 0]
    ws[...] = jnp.where(s[...] > score_thr, s[...], -jnp.inf)

    ti = _tile_iota()

    def tree_sweep(idx, valid):
        acc0 = acc1 = None
        for g in range(_G):
            gs = pl.ds(g * 8, 8)
            w_g = ws[gs, :]
            if idx is not None:
                pick_g = ((ti + g * 1024) == idx) & valid
                w_g = jnp.where(pick_g, -jnp.inf, w_g)
                ws[gs, :] = w_g
            node = (w_g, ti + g * 1024)
            if g % 2 == 0:
                acc0 = node if acc0 is None else _combine(acc0, node)
            else:
                acc1 = node if acc1 is None else _combine(acc1, node)
        return _combine(acc0, acc1)

    m_v0, i_v0 = tree_sweep(None, None)

    def round_fn(state):
        num, stop, sel, sx1, sy1, sx2, sy2, sa, m_v, i_v = state
        m = jnp.max(m_v)
        valid = m > -jnp.inf
        idx = jnp.min(jnp.where(m_v == m, i_v, _BIG))
        r = idx // _C
        c = idx - r * _C
        zero = jnp.float32(0.0)
        b0 = x1s[r, c]
        b1 = y1s[r, c]
        b2 = x2s[r, c]
        b3 = y2s[r, c]
        a = (b2 - b0) * (b3 - b1)
        nm_v, ni_v = tree_sweep(idx, valid)
        xx1 = jnp.maximum(b0, sx1)
        yy1 = jnp.maximum(b1, sy1)
        xx2 = jnp.minimum(b2, sx2)
        yy2 = jnp.minimum(b3, sy2)
        inter = (jnp.clip(xx2 - xx1, 0.0, None)
                 * jnp.clip(yy2 - yy1, 0.0, None))
        union = jnp.maximum(a + sa - inter, 1e-6)
        iou = inter / union
        hit = (iou >= iou_thr) & (ti < num)
        hitcnt = jnp.sum(jnp.where(hit, 1.0, zero))
        accepted = valid & (hitcnt == zero) & (num < _MOUT)
        slot = accepted & (ti == num)
        sel = jnp.where(slot, idx, sel)
        sx1 = jnp.where(slot, b0, sx1)
        sy1 = jnp.where(slot, b1, sy1)
        sx2 = jnp.where(slot, b2, sx2)
        sy2 = jnp.where(slot, b3, sy2)
        sa = jnp.where(slot, a, sa)
        num = num + accepted.astype(jnp.int32)
        stop = jnp.logical_not(valid)
        return (num, stop, sel, sx1, sy1, sx2, sy2, sa, nm_v, ni_v)

    def cond(carry):
        return jnp.logical_and(carry[0] < _MOUT, jnp.logical_not(carry[1]))

    def body(carry):
        state = carry
        for _ in range(_B):
            state = round_fn(state)
        return state

    zf = jnp.zeros((8, _C), jnp.float32)
    carry = (jnp.int32(0), jnp.bool_(False),
             jnp.zeros((8, _C), jnp.int32), zf, zf, zf, zf, zf, m_v0, i_v0)
    carry = jax.lax.while_loop(cond, body, carry)
    sel_ref[...] = carry[2]
    num_ref[0, 0] = carry[0]


def kernel(boxes, scores, max_output_size, iou_threshold, scores_threshold):
    boxes = boxes.astype(jnp.float32)
    scores = scores.astype(jnp.float32)
    n = boxes.shape[0]
    pad = _P - n
    bx = jnp.pad(boxes, ((0, pad), (0, 0)))
    planes = bx.T.reshape(4, _R, _C)
    s = jnp.pad(scores, (0, pad), constant_values=-jnp.inf).reshape(_R, _C)
    thr = jnp.stack([jnp.asarray(iou_threshold, jnp.float32),
                     jnp.asarray(scores_threshold, jnp.float32)]).reshape(2, 1)

    sel_m, num_m = pl.pallas_call(
        _nms_kernel,
        in_specs=[
            pl.BlockSpec(memory_space=pltpu.SMEM),
            pl.BlockSpec(memory_space=pltpu.SMEM),
            pl.BlockSpec(memory_space=pltpu.SMEM),
            pl.BlockSpec(memory_space=pltpu.SMEM),
            pl.BlockSpec(memory_space=pltpu.SMEM),
            pl.BlockSpec(memory_space=pltpu.VMEM),
        ],
        out_specs=[
            pl.BlockSpec(memory_space=pltpu.VMEM),
            pl.BlockSpec(memory_space=pltpu.SMEM),
        ],
        out_shape=[
            jax.ShapeDtypeStruct((8, _C), jnp.int32),
            jax.ShapeDtypeStruct((1, 1), jnp.int32),
        ],
        scratch_shapes=[
            pltpu.VMEM((_R, _C), jnp.float32),
        ],
    )(thr, planes[0], planes[1], planes[2], planes[3], s)

    sel = sel_m.reshape(-1)[:_MOUT]
    num = jnp.minimum(num_m[0, 0], jnp.asarray(max_output_size, jnp.int32))
    return (sel, num)


# speedup vs baseline: 1.3175x; 1.3137x over previous
import functools

import jax
import jax.numpy as jnp
from jax import lax
from jax.experimental import pallas as pl
from jax.experimental.pallas import tpu as pltpu
from jax.experimental.pallas import tpu_sc as plsc

_N = 20000
_PAD = 20480
_NC1 = _PAD // 16
_NC2 = _NC1 // 16
_MOUT = 200
_SLOTC = 13
_NEG = -jnp.inf
_INTERPRET = False


def _iota16():
    return jax.lax.broadcasted_iota(jnp.int32, (16,), 0)


def _extract_f(chunk, lane):
    sel = (_iota16() == lane).astype(jnp.float32)
    return jnp.sum(chunk * sel)


def _first_lane(mask):
    off = jnp.logical_not(mask).astype(jnp.int32) * 99
    return jnp.min(_iota16() + off)


def _sc_body(x1h, y1h, x2h, y2h, sh, thrh, selh, numh,
             x1, y1, x2, y2, ws, l1, l2, thrv, selv, numv,
             sx1, sy1, sx2, sy2, sa):
    wid = lax.axis_index("s") * 2 + lax.axis_index("c")

    @pl.when(wid == 0)
    def _work():
        pltpu.sync_copy(x1h, x1)
        pltpu.sync_copy(y1h, y1)
        pltpu.sync_copy(x2h, x2)
        pltpu.sync_copy(y2h, y2)
        pltpu.sync_copy(sh, ws)
        pltpu.sync_copy(thrh, thrv)
        it = _iota16()
        thrc = thrv[...]
        iou_thr = jnp.sum(thrc * (it == 0).astype(jnp.float32))
        score_thr = jnp.sum(thrc * (it == 1).astype(jnp.float32))

        zi = jnp.zeros((16,), jnp.int32)
        for k in range(_SLOTC + 3):
            selv[pl.ds(k * 16, 16)] = zi

        def initb(j, _):
            acc = jnp.full((16,), _NEG, jnp.float32)
            for k in range(16):
                cs = pl.ds(j * 256 + k * 16, 16)
                w = ws[cs]
                w = jnp.where(w > score_thr, w, _NEG)
                ws[cs] = w
                mk = jnp.max(w)
                acc = jnp.where(it == k, mk, acc)
            l1[pl.ds(j * 16, 16)] = acc
            return 0

        lax.fori_loop(0, _NC2, initb, 0)

        def initc(c, _):
            acc = jnp.full((16,), _NEG, jnp.float32)
            for k in range(16):
                mk = jnp.max(l1[pl.ds(c * 256 + k * 16, 16)])
                acc = jnp.where(it == k, mk, acc)
            l2[pl.ds(c * 16, 16)] = acc
            return 0

        lax.fori_loop(0, _NC2 // 16, initc, 0)

        def cond(carry):
            num, stop = carry
            return jnp.logical_and(num < _MOUT, jnp.logical_not(stop))

        def body(carry):
            num, stop = carry
            m = jnp.float32(_NEG)
            c_best = jnp.int32(0)
            for c in range(_NC2 // 16):
                mc = jnp.max(l2[pl.ds(c * 16, 16)])
                take = mc > m
                c_best = jnp.where(take, jnp.int32(c), c_best)
                m = jnp.maximum(mc, m)
            valid = m > _NEG
            c16 = pl.multiple_of(c_best * 16, 8)
            l2c = l2[pl.ds(c16, 16)]
            lane2 = _first_lane(l2c == m)
            j2 = c_best * 16 + lane2
            j216 = pl.multiple_of(j2 * 16, 8)
            l1c = l1[pl.ds(j216, 16)]
            lane1 = _first_lane(l1c == m)
            j1 = j2 * 16 + lane1
            j116 = pl.multiple_of(j1 * 16, 8)
            wchunk = ws[pl.ds(j116, 16)]
            lane0 = _first_lane(wchunk == m)
            idx = j1 * 16 + lane0

            b0 = _extract_f(x1[pl.ds(j116, 16)], lane0)
            b1 = _extract_f(y1[pl.ds(j116, 16)], lane0)
            b2 = _extract_f(x2[pl.ds(j116, 16)], lane0)
            b3 = _extract_f(y2[pl.ds(j116, 16)], lane0)
            a = (b2 - b0) * (b3 - b1)

            hitacc = jnp.zeros((16,), jnp.float32)
            for k in range(_SLOTC):
                cs = pl.ds(k * 16, 16)
                xx1 = jnp.maximum(b0, sx1[cs])
                yy1 = jnp.maximum(b1, sy1[cs])
                xx2 = jnp.minimum(b2, sx2[cs])
                yy2 = jnp.minimum(b3, sy2[cs])
                inter = (jnp.clip(xx2 - xx1, 0.0, None)
                         * jnp.clip(yy2 - yy1, 0.0, None))
                union = jnp.maximum(a + sa[cs] - inter, 1e-6)
                iou = inter / union
                hit = (iou >= iou_thr) & ((k * 16 + it) < num)
                hitacc = jnp.maximum(hitacc,
                                     jnp.where(hit, 1.0, jnp.float32(0.0)))
            hitcnt = jnp.max(hitacc)
            accepted = valid & (hitcnt == 0.0) & (num < _MOUT)

            @pl.when(accepted)
            def _store_slot():
                ks = pl.ds(pl.multiple_of((num // 16) * 16, 8), 16)
                lm = it == (num - (num // 16) * 16)
                selv[ks] = jnp.where(lm, idx, selv[ks])
                sx1[ks] = jnp.where(lm, b0, sx1[ks])
                sy1[ks] = jnp.where(lm, b1, sy1[ks])
                sx2[ks] = jnp.where(lm, b2, sx2[ks])
                sy2[ks] = jnp.where(lm, b3, sy2[ks])
                sa[ks] = jnp.where(lm, a, sa[ks])

            @pl.when(valid)
            def _remove():
                w2 = jnp.where(it == lane0, _NEG, wchunk)
                ws[pl.ds(j116, 16)] = w2
                nm1 = jnp.max(w2)
                l1c2 = jnp.where(it == lane1, nm1, l1c)
                l1[pl.ds(j216, 16)] = l1c2
                nm2 = jnp.max(l1c2)
                l2[pl.ds(c16, 16)] = jnp.where(it == lane2, nm2, l2c)

            num = num + accepted.astype(jnp.int32)
            stop = jnp.logical_not(valid)
            return (num, stop)

        num, _ = lax.while_loop(cond, body, (jnp.int32(0), jnp.bool_(False)))
        numv[...] = jnp.where(it == 0, num, 0)
        pltpu.sync_copy(selv, selh)
        pltpu.sync_copy(numv, numh)


def kernel(boxes, scores, max_output_size, iou_threshold, scores_threshold):
    boxes = boxes.astype(jnp.float32)
    scores = scores.astype(jnp.float32)
    n = boxes.shape[0]
    pad = _PAD - n
    bx = jnp.pad(boxes, ((0, pad), (0, 0)))
    s = jnp.pad(scores, (0, pad), constant_values=-jnp.inf)
    thr = jnp.zeros((16,), jnp.float32)
    thr = thr.at[0].set(jnp.asarray(iou_threshold, jnp.float32))
    thr = thr.at[1].set(jnp.asarray(scores_threshold, jnp.float32))

    mesh = plsc.VectorSubcoreMesh(core_axis_name="c", subcore_axis_name="s")
    f = functools.partial(
        pl.kernel, mesh=mesh,
        compiler_params=pltpu.CompilerParams(needs_layout_passes=False),
        interpret=_INTERPRET,
        out_type=[
            jax.ShapeDtypeStruct(((_SLOTC + 3) * 16,), jnp.int32),
            jax.ShapeDtypeStruct((16,), jnp.int32),
        ],
        scratch_types=[
            pltpu.VMEM((_PAD,), jnp.float32),
            pltpu.VMEM((_PAD,), jnp.float32),
            pltpu.VMEM((_PAD,), jnp.float32),
            pltpu.VMEM((_PAD,), jnp.float32),
            pltpu.VMEM((_PAD,), jnp.float32),
            pltpu.VMEM((_NC1,), jnp.float32),
            pltpu.VMEM((_NC2,), jnp.float32),
            pltpu.VMEM((16,), jnp.float32),
            pltpu.VMEM(((_SLOTC + 3) * 16,), jnp.int32),
            pltpu.VMEM((16,), jnp.int32),
            pltpu.VMEM((_SLOTC * 16,), jnp.float32),
            pltpu.VMEM((_SLOTC * 16,), jnp.float32),
            pltpu.VMEM((_SLOTC * 16,), jnp.float32),
            pltpu.VMEM((_SLOTC * 16,), jnp.float32),
            pltpu.VMEM((_SLOTC * 16,), jnp.float32),
        ],
    )(_sc_body)
    sel_m, num_m = f(bx[:, 0], bx[:, 1], bx[:, 2], bx[:, 3], s, thr)

    sel = sel_m[:_MOUT]
    num = jnp.minimum(num_m[0], jnp.asarray(max_output_size, jnp.int32))
    return (sel, num)
